# Initial kernel scaffold; baseline (speedup 1.0000x reference)
import jax
import jax.numpy as jnp
from jax.experimental import pallas as pl


def kernel(x, y, pos_id):
    raise NotImplementedError("write your pallas kernel here")


# speedup vs baseline: 1.9390x; 1.9390x over previous
import functools

import jax
import jax.numpy as jnp
from jax import lax
from jax.experimental import pallas as pl
from jax.experimental.pallas import tpu as pltpu
from jax.experimental.pallas import tpu_sc as plsc

_NC = 2
_NS = 16
_NW = _NC * _NS


@functools.partial(jax.jit, static_argnums=(2, 3))
def _sc_gather(table, idx, v, d):
    b = idx.shape[0]
    bpw = b // _NW
    mesh = plsc.VectorSubcoreMesh(core_axis_name="c", subcore_axis_name="s")

    @functools.partial(
        pl.kernel,
        mesh=mesh,
        out_type=jax.ShapeDtypeStruct((b, d), jnp.float32),
        scratch_types=[
            pltpu.VMEM((bpw,), jnp.int32),
            pltpu.VMEM((bpw, d), jnp.float32),
            pltpu.SemaphoreType.DMA,
        ],
    )
    def k(table_hbm, idx_hbm, out_hbm, idx_v, rows_v, sem):
        wid = lax.axis_index("s") * _NC + lax.axis_index("c")
        base = wid * bpw
        pltpu.sync_copy(idx_hbm.at[pl.ds(base, bpw)], idx_v)
        pltpu.async_copy(table_hbm.at[idx_v], rows_v, sem).wait()
        pltpu.sync_copy(rows_v, out_hbm.at[pl.ds(base, bpw)])

    return k(table, idx)


def _mul_body(g_ref, y_ref, o_ref):
    g = g_ref[...]
    yv = y_ref[...]
    o_ref[...] = g[:, None, :] * yv[None, :, :]


@functools.partial(jax.jit, static_argnums=(2,))
def _tc_mul(g, y, bs):
    b, d = g.shape
    h = y.shape[0]
    grid = (b // bs,)
    return pl.pallas_call(
        _mul_body,
        grid=grid,
        in_specs=[
            pl.BlockSpec((bs, d), lambda i: (i, 0)),
            pl.BlockSpec((h, d), lambda i: (0, 0)),
        ],
        out_specs=pl.BlockSpec((bs, h, d), lambda i: (i, 0, 0)),
        out_shape=jax.ShapeDtypeStruct((b, h, d), jnp.float32),
    )(g, y)


def kernel(x, y, pos_id):
    v, d = x.shape[2], x.shape[3]
    h = y.shape[1]
    b = pos_id.shape[0]
    table = x.reshape(v, d)
    idx = pos_id.reshape(b)
    g = _sc_gather(table, idx, v, d)
    out = _tc_mul(g, y.reshape(h, d), 256)
    return out.reshape(b, h, 1, d)
